# baseline (speedup 1.0000x reference)
import functools

import jax
import jax.numpy as jnp
from jax import lax
from jax.experimental import pallas as pl
from jax.experimental.pallas import tpu as pltpu
from jax.experimental.pallas import tpu_sc as plsc

USER_N = 2000
NODE_N = 10000
CH = 128
HCH = 64
LAYERS = 3
E = 320000
BATCHU = 1024

NT = 16
NPAD = 10240
RPT = NPAD // NT
SUBR = 64
NSUB = RPT // SUBR
EBATCH = 128
SBB = 8
NSB = 20
EB = SBB * NSB
EP = EB * EBATCH * NT
TRASH = NPAD - 1


def _rsqrt16(d):
    u = lax.bitcast_convert_type(d, jnp.int32)
    magic = jnp.full((16,), 0x5F3759DF, jnp.int32)
    h = lax.bitcast_convert_type(magic - lax.shift_right_logical(u, 1), jnp.float32)
    half = d * 0.5
    for _ in range(3):
        h = h * (1.5 - half * h * h)
    return jnp.where(d > 0.5, h, jnp.zeros((16,), jnp.float32))


def _sc_body(srcp, dstp, xh, uidx, zeros_hbm, fin, uemb,
             y_sh, acc_sh, f_hbm, dis_hbm,
             sb_src, sb_dst, ebuf0, ebuf1, sem_s0, sem_s1,
             dbuf16, uidxbuf):
    rowbuf = ebuf1.at[pl.ds(0, SUBR)]
    fbuf = ebuf0.at[pl.ds(0, SUBR)]
    ci = lax.axis_index("c")
    t = lax.axis_index("s")
    row0 = t * RPT
    ones16 = jnp.ones((16,), jnp.float32)

    def fill(ref, nrows, val16):
        def body(r, _):
            for c in range(4):
                ref[r, pl.ds(c * 16, 16)] = val16
            return 0
        lax.fori_loop(0, nrows, body, 0)

    with jax.named_scope("p0_zero"):
        pltpu.sync_copy(zeros_hbm, acc_sh.at[pl.ds(row0, RPT)])
        fill(ebuf0, EBATCH, ones16)
        plsc.subcore_barrier()

    def deg_super(sb, _):
        pltpu.sync_copy(dstp.at[t, pl.ds(sb * SBB, SBB)], sb_dst)

        def deg_fire(j, _):
            pltpu.async_copy(ebuf0, acc_sh.at[sb_dst.at[j]], sem_s0, add=True)
            return 0
        lax.fori_loop(0, SBB, deg_fire, 0)

        def deg_drain(j, _):
            pltpu.make_async_copy(ebuf0, acc_sh.at[sb_dst.at[j]], sem_s0).wait()
            return 0
        lax.fori_loop(0, SBB, deg_drain, 0)
        return 0
    with jax.named_scope("p0_deg"):
        lax.fori_loop(0, NSB, deg_super, 0)
        plsc.subcore_barrier()

    def phase0_sub(sub, _):
        r0 = row0 + sub * SUBR
        pltpu.sync_copy(acc_sh.at[pl.ds(r0, SUBR)], rowbuf)

        def dis_step(r2, _):
            for r in (2 * r2, 2 * r2 + 1):
                dbuf16[r, pl.ds(0, 16)] = _rsqrt16(rowbuf[r, pl.ds(0, 16)])
            return 0
        lax.fori_loop(0, SUBR // 2, dis_step, 0)
        pltpu.sync_copy(dbuf16, dis_hbm.at[ci, pl.ds(r0, SUBR)])

        pltpu.sync_copy(xh.at[ci, pl.ds(r0, SUBR)], rowbuf)

        def scale0(r2, _):
            for r in (2 * r2, 2 * r2 + 1):
                dv = dbuf16[r, pl.ds(0, 16)]
                for c in range(4):
                    cs = pl.ds(c * 16, 16)
                    rowbuf[r, cs] = rowbuf[r, cs] * dv
            return 0
        lax.fori_loop(0, SUBR // 2, scale0, 0)
        pltpu.sync_copy(rowbuf, y_sh.at[pl.ds(r0, SUBR)])
        return 0
    with jax.named_scope("p0_y0"):
        lax.fori_loop(0, NSUB, phase0_sub, 0)
        pltpu.sync_copy(zeros_hbm, acc_sh.at[pl.ds(row0, RPT)])
        plsc.subcore_barrier()

    def edge_phase():
        def edge_super(sb, _):
            pltpu.sync_copy(srcp.at[t, pl.ds(sb * SBB, SBB)], sb_src)
            pltpu.sync_copy(dstp.at[t, pl.ds(sb * SBB, SBB)], sb_dst)

            bufs = (ebuf0, ebuf1)
            sems = (sem_s0, sem_s1)

            def edge_pair(q, _):
                for k in range(2):
                    j = 2 * q + k

                    @pl.when(q > 0)
                    def _(k=k, j=j):
                        pltpu.make_async_copy(
                            bufs[k], acc_sh.at[sb_dst.at[j - 2]],
                            sems[k]).wait()
                    pltpu.sync_copy(y_sh.at[sb_src.at[j]], bufs[k])
                    pltpu.async_copy(bufs[k], acc_sh.at[sb_dst.at[j]],
                                     sems[k], add=True)
                return 0
            lax.fori_loop(0, SBB // 2, edge_pair, 0)
            for k in range(2):
                pltpu.make_async_copy(
                    bufs[k], acc_sh.at[sb_dst.at[SBB - 2 + k]],
                    sems[k]).wait()
            return 0
        lax.fori_loop(0, NSB, edge_super, 0)

    for l in range(LAYERS):
        with jax.named_scope(f"edges{l}"):
            edge_phase()
            plsc.subcore_barrier()

        last = l == LAYERS - 1

        def row_pass(sub, _):
            r0 = row0 + sub * SUBR
            pltpu.sync_copy(acc_sh.at[pl.ds(r0, SUBR)], rowbuf)
            pltpu.sync_copy(dis_hbm.at[ci, pl.ds(r0, SUBR)], dbuf16)
            if l == 0:
                pltpu.sync_copy(rowbuf, f_hbm.at[ci, pl.ds(r0, SUBR)])

                def upd(r2, _):
                    for r in (2 * r2, 2 * r2 + 1):
                        dv = dbuf16[r, pl.ds(0, 16)]
                        dv = dv * dv
                        for c in range(4):
                            cs = pl.ds(c * 16, 16)
                            rowbuf[r, cs] = rowbuf[r, cs] * dv
                    return 0
                lax.fori_loop(0, SUBR // 2, upd, 0)
                pltpu.sync_copy(rowbuf, y_sh.at[pl.ds(r0, SUBR)])
            elif l == 1:
                pltpu.sync_copy(f_hbm.at[ci, pl.ds(r0, SUBR)], fbuf)

                def upd1(r2, _):
                    for r in (2 * r2, 2 * r2 + 1):
                        dv = dbuf16[r, pl.ds(0, 16)]
                        dv = dv * dv
                        for c in range(4):
                            cs = pl.ds(c * 16, 16)
                            fbuf[r, cs] = (rowbuf[r, cs] - fbuf[r, cs]) * dv
                    return 0
                lax.fori_loop(0, SUBR // 2, upd1, 0)
                pltpu.sync_copy(fbuf, y_sh.at[pl.ds(r0, SUBR)])
                pltpu.sync_copy(rowbuf, f_hbm.at[ci, pl.ds(r0, SUBR)])
            else:
                pltpu.sync_copy(xh.at[ci, pl.ds(r0, SUBR)], fbuf)

                def finish(r2, _):
                    for r in (2 * r2, 2 * r2 + 1):
                        dv = dbuf16[r, pl.ds(0, 16)]
                        for c in range(4):
                            cs = pl.ds(c * 16, 16)
                            rowbuf[r, cs] = fbuf[r, cs] + rowbuf[r, cs] * dv
                    return 0
                lax.fori_loop(0, SUBR // 2, finish, 0)
                pltpu.sync_copy(rowbuf, fin.at[ci, pl.ds(r0, SUBR)])
                pltpu.sync_copy(rowbuf, y_sh.at[pl.ds(r0, SUBR)])
            return 0
        with jax.named_scope(f"rows{l}"):
            lax.fori_loop(0, NSUB, row_pass, 0)
            plsc.subcore_barrier()

    pltpu.sync_copy(uidx.at[pl.ds(t * 64, 64)], uidxbuf)
    pltpu.sync_copy(y_sh.at[uidxbuf], rowbuf)
    pltpu.sync_copy(rowbuf, uemb.at[ci, pl.ds(t * 64, 64)])


_sc_prop = functools.partial(
    pl.kernel,
    out_type=(
        jax.ShapeDtypeStruct((2, NPAD, HCH), jnp.float32),
        jax.ShapeDtypeStruct((2, BATCHU, HCH), jnp.float32),
    ),
    mesh=plsc.VectorSubcoreMesh(core_axis_name="c", subcore_axis_name="s"),
    scratch_types=[
        pltpu.VMEM_SHARED((NPAD, HCH), jnp.float32),
        pltpu.VMEM_SHARED((NPAD, HCH), jnp.float32),
        pltpu.HBM((2, NPAD, HCH), jnp.float32),
        pltpu.HBM((2, NPAD, 16), jnp.float32),
        pltpu.VMEM((SBB, EBATCH), jnp.int32),
        pltpu.VMEM((SBB, EBATCH), jnp.int32),
        pltpu.VMEM((EBATCH, HCH), jnp.float32),
        pltpu.VMEM((EBATCH, HCH), jnp.float32),
        pltpu.SemaphoreType.DMA,
        pltpu.SemaphoreType.DMA,
        pltpu.VMEM((SUBR, 16), jnp.float32),
        pltpu.VMEM((64,), jnp.int32),
    ],
)(_sc_body)


ITEM_N = NODE_N - USER_N
UB = 128


def _mm_body(c0, c1, u0, u1, o):
    dn = (((1,), (1,)), ((), ()))
    o[...] = (
        lax.dot_general(u0[0], c0[0], dn, preferred_element_type=jnp.float32)
        + lax.dot_general(u1[0], c1[0], dn, preferred_element_type=jnp.float32))


def _tc_logits(cand, uemb):
    return pl.pallas_call(
        _mm_body,
        grid=(BATCHU // UB,),
        in_specs=[
            pl.BlockSpec((1, ITEM_N, HCH), lambda i: (0, 0, 0)),
            pl.BlockSpec((1, ITEM_N, HCH), lambda i: (1, 0, 0)),
            pl.BlockSpec((1, UB, HCH), lambda i: (0, i, 0)),
            pl.BlockSpec((1, UB, HCH), lambda i: (1, i, 0)),
        ],
        out_specs=pl.BlockSpec((UB, ITEM_N), lambda i: (i, 0)),
        out_shape=jax.ShapeDtypeStruct((BATCHU, ITEM_N), jnp.float32),
    )(cand, cand, uemb, uemb)


def kernel(edge_index, user_idx, seq, id_emb):
    del seq
    edge_index = edge_index.astype(jnp.int32)
    pad = jnp.full((EP - E,), TRASH, jnp.int32)
    srcp = jnp.concatenate([edge_index[0], pad]).reshape(NT, EB, EBATCH)
    dstp = jnp.concatenate([edge_index[1], pad]).reshape(NT, EB, EBATCH)
    x0 = jnp.pad(id_emb, ((0, NPAD - NODE_N), (0, 0)))
    xh = jnp.stack([x0[:, :HCH], x0[:, HCH:]])
    zeros_blk = jnp.zeros((RPT, HCH), jnp.float32)
    fin, uemb = _sc_prop(srcp, dstp, xh, user_idx.astype(jnp.int32), zeros_blk)
    return _tc_logits(fin[:, USER_N:NODE_N], uemb)


# speedup vs baseline: 1.0878x; 1.0878x over previous
import functools

import jax
import jax.numpy as jnp
from jax import lax
from jax.experimental import pallas as pl
from jax.experimental.pallas import tpu as pltpu
from jax.experimental.pallas import tpu_sc as plsc

USER_N = 2000
NODE_N = 10000
CH = 128
HCH = 64
LAYERS = 3
E = 320000
BATCHU = 1024

NT = 16
NPAD = 10240
RPT = NPAD // NT
SUBR = 64
NSUB = RPT // SUBR
EBATCH = 64
SBB = 32
NSB = 10
EB = SBB * NSB
EP = EB * EBATCH * NT
TRASH = NPAD - 1


def _rsqrt16(d):
    u = lax.bitcast_convert_type(d, jnp.int32)
    magic = jnp.full((16,), 0x5F3759DF, jnp.int32)
    h = lax.bitcast_convert_type(magic - lax.shift_right_logical(u, 1), jnp.float32)
    half = d * 0.5
    for _ in range(3):
        h = h * (1.5 - half * h * h)
    return jnp.where(d > 0.5, h, jnp.zeros((16,), jnp.float32))


def _sc_body(srcp, dstp, xh, uidx, zeros_hbm, fin, uemb,
             y_sh, acc_sh, f_hbm, dis_hbm,
             sb_src, sb_dst, ebuf0, ebuf1, ebuf2, ebuf3,
             sem_s0, sem_s1, sem_s2, sem_s3,
             dbuf16):
    rowbuf = ebuf1
    fbuf = ebuf0
    ci = lax.axis_index("c")
    t = lax.axis_index("s")
    row0 = t * RPT
    ones16 = jnp.ones((16,), jnp.float32)

    def fill(ref, nrows, val16):
        def body(r, _):
            for c in range(4):
                ref[r, pl.ds(c * 16, 16)] = val16
            return 0
        lax.fori_loop(0, nrows, body, 0)

    with jax.named_scope("p0_zero"):
        pltpu.sync_copy(zeros_hbm, acc_sh.at[pl.ds(row0, RPT)])
        fill(ebuf0, EBATCH, ones16)
        plsc.subcore_barrier()

    def deg_super(sb, _):
        pltpu.sync_copy(dstp.at[t, pl.ds(sb * SBB, SBB)], sb_dst)

        def deg_fire(j, _):
            pltpu.async_copy(ebuf0, acc_sh.at[sb_dst.at[j]], sem_s0, add=True)
            return 0
        lax.fori_loop(0, SBB, deg_fire, 0)

        def deg_drain(j, _):
            pltpu.make_async_copy(ebuf0, acc_sh.at[sb_dst.at[j]], sem_s0).wait()
            return 0
        lax.fori_loop(0, SBB, deg_drain, 0)
        return 0
    with jax.named_scope("p0_deg"):
        lax.fori_loop(0, NSB, deg_super, 0)
        plsc.subcore_barrier()

    def phase0_sub(sub, _):
        r0 = row0 + sub * SUBR
        pltpu.sync_copy(acc_sh.at[pl.ds(r0, SUBR)], rowbuf)

        def dis_step(r2, _):
            for r in (2 * r2, 2 * r2 + 1):
                dbuf16[r, pl.ds(0, 16)] = _rsqrt16(rowbuf[r, pl.ds(0, 16)])
            return 0
        lax.fori_loop(0, SUBR // 2, dis_step, 0)
        pltpu.sync_copy(dbuf16, dis_hbm.at[ci, pl.ds(r0, SUBR)])

        pltpu.sync_copy(xh.at[ci, pl.ds(r0, SUBR)], rowbuf)

        def scale0(r2, _):
            for r in (2 * r2, 2 * r2 + 1):
                dv = dbuf16[r, pl.ds(0, 16)]
                for c in range(4):
                    cs = pl.ds(c * 16, 16)
                    rowbuf[r, cs] = rowbuf[r, cs] * dv
            return 0
        lax.fori_loop(0, SUBR // 2, scale0, 0)
        pltpu.sync_copy(rowbuf, y_sh.at[pl.ds(r0, SUBR)])
        return 0
    with jax.named_scope("p0_y0"):
        lax.fori_loop(0, NSUB, phase0_sub, 0)
        pltpu.sync_copy(zeros_hbm, acc_sh.at[pl.ds(row0, RPT)])
        plsc.subcore_barrier()

    def edge_phase():
        def edge_super(sb, _):
            pltpu.sync_copy(srcp.at[t, pl.ds(sb * SBB, SBB)], sb_src)
            pltpu.sync_copy(dstp.at[t, pl.ds(sb * SBB, SBB)], sb_dst)

            bufs = (ebuf0, ebuf1, ebuf2, ebuf3)
            sems = (sem_s0, sem_s1, sem_s2, sem_s3)

            def edge_quad(q, _):
                for k in range(4):
                    j = 4 * q + k

                    @pl.when(q > 0)
                    def _(k=k, j=j):
                        pltpu.make_async_copy(
                            bufs[k], acc_sh.at[sb_dst.at[j - 4]],
                            sems[k]).wait()
                    pltpu.sync_copy(y_sh.at[sb_src.at[j]], bufs[k])
                    pltpu.async_copy(bufs[k], acc_sh.at[sb_dst.at[j]],
                                     sems[k], add=True)
                return 0
            lax.fori_loop(0, SBB // 4, edge_quad, 0)
            for k in range(4):
                pltpu.make_async_copy(
                    bufs[k], acc_sh.at[sb_dst.at[SBB - 4 + k]],
                    sems[k]).wait()
            return 0
        lax.fori_loop(0, NSB, edge_super, 0)

    for l in range(LAYERS):
        with jax.named_scope(f"edges{l}"):
            edge_phase()
            plsc.subcore_barrier()

        last = l == LAYERS - 1

        def row_pass(sub, _):
            r0 = row0 + sub * SUBR
            pltpu.sync_copy(acc_sh.at[pl.ds(r0, SUBR)], rowbuf)
            pltpu.sync_copy(dis_hbm.at[ci, pl.ds(r0, SUBR)], dbuf16)
            if l == 0:
                pltpu.sync_copy(rowbuf, f_hbm.at[ci, pl.ds(r0, SUBR)])

                def upd(r2, _):
                    for r in (2 * r2, 2 * r2 + 1):
                        dv = dbuf16[r, pl.ds(0, 16)]
                        dv = dv * dv
                        for c in range(4):
                            cs = pl.ds(c * 16, 16)
                            rowbuf[r, cs] = rowbuf[r, cs] * dv
                    return 0
                lax.fori_loop(0, SUBR // 2, upd, 0)
                pltpu.sync_copy(rowbuf, y_sh.at[pl.ds(r0, SUBR)])
            elif l == 1:
                pltpu.sync_copy(f_hbm.at[ci, pl.ds(r0, SUBR)], fbuf)

                def upd1(r2, _):
                    for r in (2 * r2, 2 * r2 + 1):
                        dv = dbuf16[r, pl.ds(0, 16)]
                        dv = dv * dv
                        for c in range(4):
                            cs = pl.ds(c * 16, 16)
                            fbuf[r, cs] = (rowbuf[r, cs] - fbuf[r, cs]) * dv
                    return 0
                lax.fori_loop(0, SUBR // 2, upd1, 0)
                pltpu.sync_copy(fbuf, y_sh.at[pl.ds(r0, SUBR)])
                pltpu.sync_copy(rowbuf, f_hbm.at[ci, pl.ds(r0, SUBR)])
            else:
                pltpu.sync_copy(xh.at[ci, pl.ds(r0, SUBR)], fbuf)

                def finish(r2, _):
                    for r in (2 * r2, 2 * r2 + 1):
                        dv = dbuf16[r, pl.ds(0, 16)]
                        for c in range(4):
                            cs = pl.ds(c * 16, 16)
                            rowbuf[r, cs] = fbuf[r, cs] + rowbuf[r, cs] * dv
                    return 0
                lax.fori_loop(0, SUBR // 2, finish, 0)
                pltpu.sync_copy(rowbuf, fin.at[ci, pl.ds(r0, SUBR)])
                pltpu.sync_copy(rowbuf, y_sh.at[pl.ds(r0, SUBR)])
            return 0
        with jax.named_scope(f"rows{l}"):
            lax.fori_loop(0, NSUB, row_pass, 0)
            plsc.subcore_barrier()

    pltpu.sync_copy(uidx.at[pl.ds(t * 64, 64)], sb_src.at[0])
    pltpu.sync_copy(y_sh.at[sb_src.at[0]], rowbuf)
    pltpu.sync_copy(rowbuf, uemb.at[ci, pl.ds(t * 64, 64)])


_sc_prop = functools.partial(
    pl.kernel,
    out_type=(
        jax.ShapeDtypeStruct((2, NPAD, HCH), jnp.float32),
        jax.ShapeDtypeStruct((2, BATCHU, HCH), jnp.float32),
    ),
    mesh=plsc.VectorSubcoreMesh(core_axis_name="c", subcore_axis_name="s"),
    scratch_types=[
        pltpu.VMEM_SHARED((NPAD, HCH), jnp.float32),
        pltpu.VMEM_SHARED((NPAD, HCH), jnp.float32),
        pltpu.HBM((2, NPAD, HCH), jnp.float32),
        pltpu.HBM((2, NPAD, 16), jnp.float32),
        pltpu.VMEM((SBB, EBATCH), jnp.int32),
        pltpu.VMEM((SBB, EBATCH), jnp.int32),
        pltpu.VMEM((EBATCH, HCH), jnp.float32),
        pltpu.VMEM((EBATCH, HCH), jnp.float32),
        pltpu.VMEM((EBATCH, HCH), jnp.float32),
        pltpu.VMEM((EBATCH, HCH), jnp.float32),
        pltpu.SemaphoreType.DMA,
        pltpu.SemaphoreType.DMA,
        pltpu.SemaphoreType.DMA,
        pltpu.SemaphoreType.DMA,
        pltpu.VMEM((SUBR, 16), jnp.float32),
    ],
)(_sc_body)


ITEM_N = NODE_N - USER_N
UB = 128


def _mm_body(c0, c1, u0, u1, o):
    dn = (((1,), (1,)), ((), ()))
    o[...] = (
        lax.dot_general(u0[0], c0[0], dn, preferred_element_type=jnp.float32)
        + lax.dot_general(u1[0], c1[0], dn, preferred_element_type=jnp.float32))


def _tc_logits(cand, uemb):
    return pl.pallas_call(
        _mm_body,
        grid=(BATCHU // UB,),
        in_specs=[
            pl.BlockSpec((1, ITEM_N, HCH), lambda i: (0, 0, 0)),
            pl.BlockSpec((1, ITEM_N, HCH), lambda i: (1, 0, 0)),
            pl.BlockSpec((1, UB, HCH), lambda i: (0, i, 0)),
            pl.BlockSpec((1, UB, HCH), lambda i: (1, i, 0)),
        ],
        out_specs=pl.BlockSpec((UB, ITEM_N), lambda i: (i, 0)),
        out_shape=jax.ShapeDtypeStruct((BATCHU, ITEM_N), jnp.float32),
    )(cand, cand, uemb, uemb)


def kernel(edge_index, user_idx, seq, id_emb):
    del seq
    edge_index = edge_index.astype(jnp.int32)
    pad = jnp.full((EP - E,), TRASH, jnp.int32)
    srcp = jnp.concatenate([edge_index[0], pad]).reshape(NT, EB, EBATCH)
    dstp = jnp.concatenate([edge_index[1], pad]).reshape(NT, EB, EBATCH)
    x0 = jnp.pad(id_emb, ((0, NPAD - NODE_N), (0, 0)))
    xh = jnp.stack([x0[:, :HCH], x0[:, HCH:]])
    zeros_blk = jnp.zeros((RPT, HCH), jnp.float32)
    fin, uemb = _sc_prop(srcp, dstp, xh, user_idx.astype(jnp.int32), zeros_blk)
    return _tc_logits(fin[:, USER_N:NODE_N], uemb)
